# initial kernel scaffold (unmeasured)
import jax
import jax.numpy as jnp
from jax import lax
from jax.experimental import pallas as pl
from jax.experimental.pallas import tpu as pltpu

N_DEV = 4
B = 2
SQ = 512
SKV = 512
HQ = 8
DH = 64
DM = 768
DQK = HQ * DH
BLK = 64
SCALE = 0.125
NEG = -1e9


def kernel(x, Wq, K_ext, V_ext, Wo):
    def body(x_ref, wq_ref, k_ref, v_ref, wo_ref, out_ref,
             kbuf, vbuf, ksend, krecv, vsend, vrecv):
        my = lax.axis_index("i")

        barrier = pltpu.get_barrier_semaphore()
        for r in range(1, N_DEV):
            peer = lax.rem(my + r, N_DEV)
            pl.semaphore_signal(barrier, inc=1, device_id=(peer,),
                                device_id_type=pl.DeviceIdType.MESH)
        pl.semaphore_wait(barrier, N_DEV - 1)

        rdmas = []
        for r in range(1, N_DEV):
            dst = lax.rem(my + r, N_DEV)
            kr = pltpu.make_async_remote_copy(
                src_ref=k_ref, dst_ref=kbuf.at[r - 1],
                send_sem=ksend.at[r - 1], recv_sem=krecv.at[r - 1],
                device_id=(dst,), device_id_type=pl.DeviceIdType.MESH)
            vr = pltpu.make_async_remote_copy(
                src_ref=v_ref, dst_ref=vbuf.at[r - 1],
                send_sem=vsend.at[r - 1], recv_sem=vrecv.at[r - 1],
                device_id=(dst,), device_id_type=pl.DeviceIdType.MESH)
            kr.start()
            vr.start()
            rdmas.append((kr, vr))

        xv = x_ref[...].reshape(B * SQ, DM)
        q = lax.dot_general(xv, wq_ref[...], (((1,), (0,)), ((), ())),
                            preferred_element_type=jnp.float32)
        q = q.reshape(B, SQ, HQ, DH)

        qblk = (my * SQ + lax.broadcasted_iota(jnp.int32, (SQ, SKV), 0)) // BLK
        kj = lax.broadcasted_iota(jnp.int32, (SQ, SKV), 1)
        masks = []
        for s in range(N_DEV):
            c = lax.rem(my - s + N_DEV, N_DEV)
            masks.append((c * SKV + kj) // BLK <= qblk)

        for kr, vr in rdmas:
            kr.wait_recv()
            vr.wait_recv()
        for kr, vr in rdmas:
            kr.wait_send()
            vr.wait_send()

        outs = []
        for b in range(B):
            kvals = [k_ref[b]] + [kbuf[s - 1, b] for s in range(1, N_DEV)]
            vvals = [v_ref[b]] + [vbuf[s - 1, b] for s in range(1, N_DEV)]
            ctx_h = []
            for h in range(HQ):
                qbh = q[b, :, h, :]
                sc = []
                for s in range(N_DEV):
                    scs = lax.dot_general(
                        qbh, kvals[s][:, h, :], (((1,), (1,)), ((), ())),
                        preferred_element_type=jnp.float32) * SCALE
                    sc.append(jnp.where(masks[s], scs, NEG))
                scores = jnp.concatenate(sc, axis=1)
                m = jnp.max(scores, axis=1, keepdims=True)
                w = jnp.exp(scores - m)
                w = w / jnp.sum(w, axis=1, keepdims=True)
                vcat = jnp.concatenate(
                    [vvals[s][:, h, :] for s in range(N_DEV)], axis=0)
                ctx_h.append(lax.dot_general(
                    w, vcat, (((1,), (0,)), ((), ())),
                    preferred_element_type=jnp.float32))
            outs.append(jnp.concatenate(ctx_h, axis=1))

        ctx = jnp.stack(outs, axis=0).reshape(B * SQ, DQK)
        o = lax.dot_general(ctx, wo_ref[...], (((1,), (0,)), ((), ())),
                            preferred_element_type=jnp.float32)
        out_ref[...] = o.reshape(B, SQ, DM)

    return pl.pallas_call(
        body,
        out_shape=jax.ShapeDtypeStruct((B, SQ, DM), jnp.float32),
        in_specs=[pl.BlockSpec(memory_space=pltpu.VMEM)] * 5,
        out_specs=pl.BlockSpec(memory_space=pltpu.VMEM),
        scratch_shapes=[
            pltpu.VMEM((N_DEV - 1, B, SKV, HQ, DH), jnp.float32),
            pltpu.VMEM((N_DEV - 1, B, SKV, HQ, DH), jnp.float32),
            pltpu.SemaphoreType.DMA((N_DEV - 1,)),
            pltpu.SemaphoreType.DMA((N_DEV - 1,)),
            pltpu.SemaphoreType.DMA((N_DEV - 1,)),
            pltpu.SemaphoreType.DMA((N_DEV - 1,)),
        ],
        compiler_params=pltpu.CompilerParams(collective_id=0),
    )(x, Wq, K_ext, V_ext, Wo)


# baseline (device time: 252593 ns/iter reference)
import jax
import jax.numpy as jnp
from jax import lax
from jax.experimental import pallas as pl
from jax.experimental.pallas import tpu as pltpu

N_DEV = 4
B = 2
SQ = 512
SKV = 512
HQ = 8
DH = 64
DM = 768
DQK = HQ * DH
BLK = 64
SCALE = 0.125
NEG = -1e9


def kernel(x, Wq, K_ext, V_ext, Wo):
    def body(x_ref, wq_ref, k_ref, v_ref, wo_ref, out_ref,
             kbuf, vbuf, ksend, krecv, vsend, vrecv):
        my = lax.axis_index("i")

        barrier = pltpu.get_barrier_semaphore()
        for r in range(1, N_DEV):
            peer = lax.rem(my + r, N_DEV)
            pl.semaphore_signal(barrier, inc=1, device_id=(peer,),
                                device_id_type=pl.DeviceIdType.MESH)
        pl.semaphore_wait(barrier, N_DEV - 1)

        rdmas = []
        for r in range(1, N_DEV):
            dst = lax.rem(my + r, N_DEV)
            kr = pltpu.make_async_remote_copy(
                src_ref=k_ref, dst_ref=kbuf.at[r - 1],
                send_sem=ksend.at[r - 1], recv_sem=krecv.at[r - 1],
                device_id=(dst,), device_id_type=pl.DeviceIdType.MESH)
            vr = pltpu.make_async_remote_copy(
                src_ref=v_ref, dst_ref=vbuf.at[r - 1],
                send_sem=vsend.at[r - 1], recv_sem=vrecv.at[r - 1],
                device_id=(dst,), device_id_type=pl.DeviceIdType.MESH)
            kr.start()
            vr.start()
            rdmas.append((kr, vr))

        xv = x_ref[...].reshape(B * SQ, DM)
        q = lax.dot_general(xv, wq_ref[...], (((1,), (0,)), ((), ())),
                            preferred_element_type=jnp.float32)
        q = q.reshape(B, SQ, HQ, DH)

        qblk = (my * SQ + lax.broadcasted_iota(jnp.int32, (SQ, SKV), 0)) // BLK
        kj = lax.broadcasted_iota(jnp.int32, (SQ, SKV), 1)
        masks = []
        for s in range(N_DEV):
            c = lax.rem(my - s + N_DEV, N_DEV)
            masks.append((c * SKV + kj) // BLK <= qblk)

        for kr, vr in rdmas:
            kr.wait_recv()
            vr.wait_recv()
        for kr, vr in rdmas:
            kr.wait_send()
            vr.wait_send()

        outs = []
        for b in range(B):
            kvals = [k_ref[b]] + [kbuf[s - 1, b] for s in range(1, N_DEV)]
            vvals = [v_ref[b]] + [vbuf[s - 1, b] for s in range(1, N_DEV)]
            ctx_h = []
            for h in range(HQ):
                qbh = q[b, :, h, :]
                sc = []
                for s in range(N_DEV):
                    scs = lax.dot_general(
                        qbh, kvals[s][:, h, :], (((1,), (1,)), ((), ())),
                        preferred_element_type=jnp.float32) * SCALE
                    sc.append(jnp.where(masks[s], scs, NEG))
                scores = jnp.concatenate(sc, axis=1)
                m = jnp.max(scores, axis=1, keepdims=True)
                w = jnp.exp(scores - m)
                w = w / jnp.sum(w, axis=1, keepdims=True)
                vcat = jnp.concatenate(
                    [vvals[s][:, h, :] for s in range(N_DEV)], axis=0)
                ctx_h.append(lax.dot_general(
                    w, vcat, (((1,), (0,)), ((), ())),
                    preferred_element_type=jnp.float32))
            outs.append(jnp.concatenate(ctx_h, axis=1))

        ctx = jnp.stack(outs, axis=0).reshape(B * SQ, DQK)
        o = lax.dot_general(ctx, wo_ref[...], (((1,), (0,)), ((), ())),
                            preferred_element_type=jnp.float32)
        out_ref[...] = o.reshape(B, SQ, DM)

    return pl.pallas_call(
        body,
        out_shape=jax.ShapeDtypeStruct((B, SQ, DM), jnp.float32),
        in_specs=[pl.BlockSpec(memory_space=pltpu.VMEM)] * 5,
        out_specs=pl.BlockSpec(memory_space=pltpu.VMEM),
        scratch_shapes=[
            pltpu.VMEM((N_DEV - 1, B, SKV, HQ, DH), jnp.float32),
            pltpu.VMEM((N_DEV - 1, B, SKV, HQ, DH), jnp.float32),
            pltpu.SemaphoreType.DMA((N_DEV - 1,)),
            pltpu.SemaphoreType.DMA((N_DEV - 1,)),
            pltpu.SemaphoreType.DMA((N_DEV - 1,)),
            pltpu.SemaphoreType.DMA((N_DEV - 1,)),
        ],
        compiler_params=pltpu.CompilerParams(
            collective_id=0, vmem_limit_bytes=100 * 1024 * 1024),
    )(x, Wq, K_ext, V_ext, Wo)


# device time: 165978 ns/iter; 1.5218x vs baseline; 1.5218x over previous
import jax
import jax.numpy as jnp
from jax import lax
from jax.experimental import pallas as pl
from jax.experimental.pallas import tpu as pltpu

N_DEV = 4
B = 2
SQ = 512
SKV = 512
HQ = 8
DH = 64
DM = 768
DQK = HQ * DH
BLK = 64
SCALE = 0.125


def kernel(x, Wq, K_ext, V_ext, Wo):
    Kt = jnp.transpose(K_ext, (0, 2, 1, 3))
    Vt = jnp.transpose(V_ext, (0, 2, 1, 3))

    def body(x_ref, wq_ref, kt_ref, vt_ref, wo_ref, out_ref,
             kbuf, vbuf, q_ref, acc_ref, den_ref,
             ksend, krecv, vsend, vrecv):
        my = lax.axis_index("i")
        left = lax.rem(my + N_DEV - 1, N_DEV)
        right = lax.rem(my + 1, N_DEV)

        barrier = pltpu.get_barrier_semaphore()
        for nbr in (left, right):
            pl.semaphore_signal(barrier, inc=1, device_id=(nbr,),
                                device_id_type=pl.DeviceIdType.MESH)
        pl.semaphore_wait(barrier, 2)

        def remote(src, dst, ssem, rsem, dev):
            return pltpu.make_async_remote_copy(
                src_ref=src, dst_ref=dst, send_sem=ssem, recv_sem=rsem,
                device_id=(dev,), device_id_type=pl.DeviceIdType.MESH)

        k_to_left = remote(kt_ref, kbuf.at[1], ksend.at[0], krecv.at[1], left)
        v_to_left = remote(vt_ref, vbuf.at[1], vsend.at[0], vrecv.at[1], left)
        k_to_right = remote(kt_ref, kbuf.at[0], ksend.at[1], krecv.at[0], right)
        v_to_right = remote(vt_ref, vbuf.at[0], vsend.at[1], vrecv.at[0], right)
        for r in (k_to_left, v_to_left, k_to_right, v_to_right):
            r.start()

        xv = x_ref[...].reshape(B * SQ, DM)
        q_ref[...] = lax.dot_general(xv, wq_ref[...], (((1,), (0,)), ((), ())),
                                     preferred_element_type=jnp.float32) * SCALE

        i0 = lax.broadcasted_iota(jnp.int32, (SQ, SKV), 0)
        j0 = lax.broadcasted_iota(jnp.int32, (SQ, SKV), 1)
        mask_own = ((j0 // BLK) <= (i0 // BLK)).astype(jnp.float32)
        keep_l = (my >= 1).astype(jnp.float32)
        keep_d = (my >= 2).astype(jnp.float32)
        keep_r = (my >= 3).astype(jnp.float32)

        def attend(kc, vc, mask, b, h, init=False):
            qbh = q_ref[b * SQ:(b + 1) * SQ, h * DH:(h + 1) * DH]
            s = lax.dot_general(qbh, kc, (((1,), (1,)), ((), ())),
                                preferred_element_type=jnp.float32)
            p = jnp.exp(s) * mask
            pv = lax.dot_general(p, vc, (((1,), (0,)), ((), ())),
                                 preferred_element_type=jnp.float32)
            d = jnp.sum(p, axis=1, keepdims=True)
            if init:
                acc_ref[b, h] = pv
                den_ref[b, h] = d
            else:
                acc_ref[b, h] = acc_ref[b, h] + pv
                den_ref[b, h] = den_ref[b, h] + d

        def attend_all(kc_of_bh, vc_of_bh, mask, init=False):
            for b in range(B):
                for h in range(HQ):
                    attend(kc_of_bh(b, h), vc_of_bh(b, h), mask, b, h, init)

        attend_all(lambda b, h: kt_ref[b, h], lambda b, h: vt_ref[b, h],
                   mask_own, init=True)

        k_to_right.wait_recv()
        v_to_right.wait_recv()
        k_relay_r = remote(kbuf.at[0, 0], kbuf.at[2, 0],
                           ksend.at[2], krecv.at[2], right)
        v_relay_r = remote(vbuf.at[0, 0], vbuf.at[2, 0],
                           vsend.at[2], vrecv.at[2], right)
        k_relay_r.start()
        v_relay_r.start()

        k_to_left.wait_recv()
        v_to_left.wait_recv()
        k_relay_l = remote(kbuf.at[1, 1], kbuf.at[2, 1],
                           ksend.at[3], krecv.at[3], left)
        v_relay_l = remote(vbuf.at[1, 1], vbuf.at[2, 1],
                           vsend.at[3], vrecv.at[3], left)
        k_relay_l.start()
        v_relay_l.start()

        attend_all(lambda b, h: kbuf[0, b, h], lambda b, h: vbuf[0, b, h],
                   keep_l)
        attend_all(lambda b, h: kbuf[1, b, h], lambda b, h: vbuf[1, b, h],
                   keep_r)

        k_relay_r.wait_recv()
        v_relay_r.wait_recv()
        for h in range(HQ):
            attend(kbuf[2, 0, h], vbuf[2, 0, h], keep_d, 0, h)
        k_relay_l.wait_recv()
        v_relay_l.wait_recv()
        for h in range(HQ):
            attend(kbuf[2, 1, h], vbuf[2, 1, h], keep_d, 1, h)

        for r in (k_to_left, v_to_left, k_to_right, v_to_right,
                  k_relay_r, v_relay_r, k_relay_l, v_relay_l):
            r.wait_send()

        ctxs = []
        for b in range(B):
            heads = [acc_ref[b, h] / den_ref[b, h] for h in range(HQ)]
            ctxs.append(jnp.concatenate(heads, axis=1))
        ctx = jnp.concatenate(ctxs, axis=0)
        o = lax.dot_general(ctx, wo_ref[...], (((1,), (0,)), ((), ())),
                            preferred_element_type=jnp.float32)
        out_ref[...] = o.reshape(B, SQ, DM)

    return pl.pallas_call(
        body,
        out_shape=jax.ShapeDtypeStruct((B, SQ, DM), jnp.float32),
        in_specs=[pl.BlockSpec(memory_space=pltpu.VMEM)] * 5,
        out_specs=pl.BlockSpec(memory_space=pltpu.VMEM),
        scratch_shapes=[
            pltpu.VMEM((3, B, HQ, SKV, DH), jnp.float32),
            pltpu.VMEM((3, B, HQ, SKV, DH), jnp.float32),
            pltpu.VMEM((B * SQ, DQK), jnp.float32),
            pltpu.VMEM((B, HQ, SQ, DH), jnp.float32),
            pltpu.VMEM((B, HQ, SQ, 1), jnp.float32),
            pltpu.SemaphoreType.DMA((4,)),
            pltpu.SemaphoreType.DMA((4,)),
            pltpu.SemaphoreType.DMA((4,)),
            pltpu.SemaphoreType.DMA((4,)),
        ],
        compiler_params=pltpu.CompilerParams(
            collective_id=0, vmem_limit_bytes=100 * 1024 * 1024),
    )(x, Wq, Kt, Vt, Wo)


# device time: 88788 ns/iter; 2.8449x vs baseline; 1.8694x over previous
import os

import jax
import jax.numpy as jnp
from jax import lax
from jax.experimental import pallas as pl
from jax.experimental.pallas import tpu as pltpu

SKIP_COMM = bool(int(os.environ.get("SKIP_COMM", "0")))
SKIP_COMPUTE = bool(int(os.environ.get("SKIP_COMPUTE", "0")))

N_DEV = 4
B = 2
SQ = 512
SKV = 512
HQ = 8
DH = 64
DM = 768
DQK = HQ * DH
BLK = 64
SCALE = 0.125


def kernel(x, Wq, K_ext, V_ext, Wo):
    Kt = jnp.transpose(K_ext, (0, 2, 3, 1))
    Vt = jnp.transpose(V_ext, (0, 2, 3, 1))

    def body(x_ref, wq_ref, kt_ref, vt_ref, wo_ref, out_ref,
             kbuf, vbuf, q_ref, acc_ref, den_ref,
             ksend, krecv, vsend, vrecv):
        my = lax.axis_index("i")
        left = lax.rem(my + N_DEV - 1, N_DEV)
        right = lax.rem(my + 1, N_DEV)

        def remote(src, dst, ssem, rsem, dev):
            return pltpu.make_async_remote_copy(
                src_ref=src, dst_ref=dst, send_sem=ssem, recv_sem=rsem,
                device_id=(dev,), device_id_type=pl.DeviceIdType.MESH)

        if not SKIP_COMM:
            barrier = pltpu.get_barrier_semaphore()
            for nbr in (left, right):
                pl.semaphore_signal(barrier, inc=1, device_id=(nbr,),
                                    device_id_type=pl.DeviceIdType.MESH)
            pl.semaphore_wait(barrier, 2)

            k_to_left = remote(kt_ref, kbuf.at[1], ksend.at[0], krecv.at[1], left)
            v_to_left = remote(vt_ref, vbuf.at[1], vsend.at[0], vrecv.at[1], left)
            k_to_right = remote(kt_ref, kbuf.at[0], ksend.at[1], krecv.at[0], right)
            v_to_right = remote(vt_ref, vbuf.at[0], vsend.at[1], vrecv.at[0], right)
            for r in (k_to_left, v_to_left, k_to_right, v_to_right):
                r.start()

        xv = x_ref[...].reshape(B * SQ, DM)
        q_ref[...] = lax.dot_general(
            wq_ref[...], xv, (((0,), (1,)), ((), ())),
            preferred_element_type=jnp.float32) * SCALE

        j0 = lax.broadcasted_iota(jnp.int32, (SKV, SQ), 0)
        i0 = lax.broadcasted_iota(jnp.int32, (SKV, SQ), 1)
        mask_own = ((j0 // BLK) <= (i0 // BLK)).astype(jnp.float32)
        keep_l = (my >= 1).astype(jnp.float32)
        keep_d = (my >= 2).astype(jnp.float32)
        keep_r = (my >= 3).astype(jnp.float32)

        def attend(kc, vc, mask, b, h, init=False):
            if SKIP_COMPUTE:
                if init:
                    acc_ref[b, h] = kc + vc
                    den_ref[b, h] = kc[0]
                return
            q_bh = q_ref[h * DH:(h + 1) * DH, b * SQ:(b + 1) * SQ]
            s_t = lax.dot_general(kc, q_bh, (((0,), (0,)), ((), ())),
                                  preferred_element_type=jnp.float32)
            p_t = jnp.exp(s_t) * mask
            pv_t = lax.dot_general(vc, p_t, (((1,), (0,)), ((), ())),
                                   preferred_element_type=jnp.float32)
            d = jnp.sum(p_t, axis=0)
            if init:
                acc_ref[b, h] = pv_t
                den_ref[b, h] = d
            else:
                acc_ref[b, h] = acc_ref[b, h] + pv_t
                den_ref[b, h] = den_ref[b, h] + d

        def attend_all(kc_of_bh, vc_of_bh, mask, init=False):
            for b in range(B):
                for h in range(HQ):
                    attend(kc_of_bh(b, h), vc_of_bh(b, h), mask, b, h, init)

        attend_all(lambda b, h: kt_ref[b, h], lambda b, h: vt_ref[b, h],
                   mask_own, init=True)

        if not SKIP_COMM:
            k_to_right.wait_recv()
            k_relay_r = remote(kbuf.at[0, 0], kbuf.at[2, 0],
                               ksend.at[2], krecv.at[2], right)
            k_relay_r.start()
            v_to_right.wait_recv()
            v_relay_r = remote(vbuf.at[0, 0], vbuf.at[2, 0],
                               vsend.at[2], vrecv.at[2], right)
            v_relay_r.start()

            k_to_left.wait_recv()
            k_relay_l = remote(kbuf.at[1, 1], kbuf.at[2, 1],
                               ksend.at[3], krecv.at[3], left)
            k_relay_l.start()
            v_to_left.wait_recv()
            v_relay_l = remote(vbuf.at[1, 1], vbuf.at[2, 1],
                               vsend.at[3], vrecv.at[3], left)
            v_relay_l.start()

        attend_all(lambda b, h: kbuf[0, b, h], lambda b, h: vbuf[0, b, h],
                   keep_l)
        attend_all(lambda b, h: kbuf[1, b, h], lambda b, h: vbuf[1, b, h],
                   keep_r)

        if not SKIP_COMM:
            k_relay_r.wait_recv()
            v_relay_r.wait_recv()
        for h in range(HQ):
            attend(kbuf[2, 0, h], vbuf[2, 0, h], keep_d, 0, h)
        if not SKIP_COMM:
            k_relay_l.wait_recv()
            v_relay_l.wait_recv()
        for h in range(HQ):
            attend(kbuf[2, 1, h], vbuf[2, 1, h], keep_d, 1, h)

        if not SKIP_COMM:
            for r in (k_to_left, v_to_left, k_to_right, v_to_right,
                      k_relay_r, v_relay_r, k_relay_l, v_relay_l):
                r.wait_send()

        for b in range(B):
            ctx_t = jnp.concatenate(
                [acc_ref[b, h] / den_ref[b, h][None, :] for h in range(HQ)],
                axis=0)
            out_ref[b] = lax.dot_general(
                ctx_t, wo_ref[...], (((0,), (0,)), ((), ())),
                preferred_element_type=jnp.float32)

    return pl.pallas_call(
        body,
        out_shape=jax.ShapeDtypeStruct((B, SQ, DM), jnp.float32),
        in_specs=[pl.BlockSpec(memory_space=pltpu.VMEM)] * 5,
        out_specs=pl.BlockSpec(memory_space=pltpu.VMEM),
        scratch_shapes=[
            pltpu.VMEM((3, B, HQ, DH, SKV), jnp.float32),
            pltpu.VMEM((3, B, HQ, DH, SKV), jnp.float32),
            pltpu.VMEM((DQK, B * SQ), jnp.float32),
            pltpu.VMEM((B, HQ, DH, SQ), jnp.float32),
            pltpu.VMEM((B, HQ, SQ), jnp.float32),
            pltpu.SemaphoreType.DMA((4,)),
            pltpu.SemaphoreType.DMA((4,)),
            pltpu.SemaphoreType.DMA((4,)),
            pltpu.SemaphoreType.DMA((4,)),
        ],
        compiler_params=pltpu.CompilerParams(
            collective_id=None if SKIP_COMM else 0,
            vmem_limit_bytes=100 * 1024 * 1024),
    )(x, Wq, Kt, Vt, Wo)
